# baseline (speedup 1.0000x reference)
import functools

import jax
import jax.numpy as jnp
from jax import lax
from jax.experimental import pallas as pl
from jax.experimental.pallas import tpu as pltpu
from jax.experimental.pallas import tpu_sc as plsc

_EMBED_DIM = 16
_INPUT_DIM = 64
_HIDDEN = 64
_BATCH = 16384
_NT = _BATCH // 128

_NC = 2
_NS = 16
_NW = _NC * _NS
_BPW = _BATCH // _NW
_CHUNK = 128
_NCHUNK = _BPW // _CHUNK


def _sc_gather(tableT, idx3):
    mesh = plsc.VectorSubcoreMesh(core_axis_name="c", subcore_axis_name="s")

    @functools.partial(
        pl.kernel,
        mesh=mesh,
        compiler_params=pltpu.CompilerParams(use_tc_tiling_on_sc=False,
                                             needs_layout_passes=False,
                                             skip_device_barrier=True),
        out_type=jax.ShapeDtypeStruct((_NT, _EMBED_DIM, _CHUNK), jnp.float32),
        scratch_types=[
            pltpu.VMEM((_NCHUNK, _CHUNK), jnp.int32),
            pltpu.VMEM((_NCHUNK, _EMBED_DIM, _CHUNK), jnp.float32),
            pltpu.SemaphoreType.DMA,
        ],
    )
    def gather_kernel(table_hbm, idx_hbm, out_hbm, idx_v, slab_v, sem):
        wid = lax.axis_index("s") * _NC + lax.axis_index("c")
        pltpu.sync_copy(idx_hbm.at[wid], idx_v)

        def fire(j, carry):
            for c in range(_NCHUNK):
                pltpu.make_async_copy(
                    table_hbm.at[j].at[idx_v.at[c]],
                    slab_v.at[c, j],
                    sem,
                ).start()
            return carry

        lax.fori_loop(0, _EMBED_DIM, fire, 0)
        pltpu.make_async_copy(
            out_hbm.at[pl.ds(wid * _NCHUNK, _NCHUNK)], slab_v, sem).wait()
        pltpu.sync_copy(slab_v, out_hbm.at[pl.ds(wid * _NCHUNK, _NCHUNK)])

    return gather_kernel(tableT, idx3)


_BC = 4096
_TPB = _BC // 128


def _mlp1_body(featT_ref, w1T_ref, b1c_ref, xf_ref):
    w1fT = w1T_ref[:, _EMBED_DIM:]
    xf = jnp.dot(w1fT, featT_ref[...],
                 preferred_element_type=jnp.float32) + b1c_ref[...]
    xf_ref[...] = xf.astype(jnp.bfloat16)


def _tc_mlp1(featT, w1T, b1c):
    grid = (_BATCH // _BC,)
    return pl.pallas_call(
        _mlp1_body,
        grid=grid,
        in_specs=[
            pl.BlockSpec((_INPUT_DIM, _BC), lambda i: (0, i)),
            pl.BlockSpec((_INPUT_DIM, _EMBED_DIM + _INPUT_DIM),
                         lambda i: (0, 0)),
            pl.BlockSpec((_HIDDEN, 1), lambda i: (0, 0)),
        ],
        out_specs=pl.BlockSpec((_HIDDEN, _BC), lambda i: (0, i)),
        out_shape=jax.ShapeDtypeStruct((_HIDDEN, _BATCH), jnp.bfloat16),
    )(featT, w1T, b1c)


def _mlp2_body(xf_ref, e_ref, w1T_ref, w2c_ref, b2_ref, out_ref):
    w1eT = w1T_ref[:, 0:_EMBED_DIM]
    e = e_ref[...]
    xe = jnp.concatenate(
        [jnp.dot(w1eT, e[t], preferred_element_type=jnp.float32)
         for t in range(_TPB)], axis=1)
    hT = jnp.maximum(xf_ref[...].astype(jnp.float32) + xe, 0.0)
    o = jnp.sum(hT * w2c_ref[...], axis=0) + b2_ref[0, 0]
    out_ref[...] = jnp.concatenate(
        [o.reshape(1, _BC), jnp.zeros((7, _BC), jnp.float32)], axis=0)


def _tc_mlp2(xfT, E, w1T, w2c, b2r):
    grid = (_BATCH // _BC,)
    return pl.pallas_call(
        _mlp2_body,
        grid=grid,
        in_specs=[
            pl.BlockSpec((_HIDDEN, _BC), lambda i: (0, i)),
            pl.BlockSpec((_TPB, _EMBED_DIM, _CHUNK), lambda i: (i, 0, 0)),
            pl.BlockSpec((_INPUT_DIM, _EMBED_DIM + _INPUT_DIM),
                         lambda i: (0, 0)),
            pl.BlockSpec((_HIDDEN, 1), lambda i: (0, 0)),
            pl.BlockSpec((1, 1), lambda i: (0, 0)),
        ],
        out_specs=pl.BlockSpec((8, _BC), lambda i: (0, i)),
        out_shape=jax.ShapeDtypeStruct((8, _BATCH), jnp.float32),
    )(xfT, E, w1T, w2c, b2r)


def kernel(pitcher_id, features, table, W1, b1, W2, b2):
    pid = pitcher_id.astype(jnp.int32)
    idx3 = pid.reshape(_NW, _NCHUNK, _CHUNK)
    w1T = W1.T
    E = _sc_gather(table.T, idx3)
    xfT = _tc_mlp1(features.T, w1T, b1.reshape(_HIDDEN, 1))
    out8 = _tc_mlp2(xfT, E, w1T, W2, b2.reshape(1, 1))
    return out8[:1, :].reshape(_BATCH, 1)


# speedup vs baseline: 1.0019x; 1.0019x over previous
import functools

import jax
import jax.numpy as jnp
from jax import lax
from jax.experimental import pallas as pl
from jax.experimental.pallas import tpu as pltpu
from jax.experimental.pallas import tpu_sc as plsc

_EMBED_DIM = 16
_INPUT_DIM = 64
_HIDDEN = 64
_BATCH = 16384
_NT = _BATCH // 128

_NC = 2
_NS = 16
_NW = _NC * _NS
_BPW = _BATCH // _NW
_CHUNK = 128
_NCHUNK = _BPW // _CHUNK


def _sc_gather(tableT, idx3):
    mesh = plsc.VectorSubcoreMesh(core_axis_name="c", subcore_axis_name="s")

    @functools.partial(
        pl.kernel,
        mesh=mesh,
        compiler_params=pltpu.CompilerParams(use_tc_tiling_on_sc=False,
                                             needs_layout_passes=False),
        out_type=jax.ShapeDtypeStruct((_NT, _EMBED_DIM, _CHUNK), jnp.float32),
        scratch_types=[
            pltpu.VMEM((_NCHUNK, _CHUNK), jnp.int32),
            pltpu.VMEM((_NCHUNK, _EMBED_DIM, _CHUNK), jnp.float32),
            pltpu.SemaphoreType.DMA,
        ],
    )
    def gather_kernel(table_hbm, idx_hbm, out_hbm, idx_v, slab_v, sem):
        wid = lax.axis_index("s") * _NC + lax.axis_index("c")
        pltpu.sync_copy(idx_hbm.at[wid], idx_v)

        def fire(i, carry):
            j = lax.shift_right_logical(i, 2)
            c = i & 3
            pltpu.make_async_copy(
                table_hbm.at[j].at[idx_v.at[c]],
                slab_v.at[c, j],
                sem,
            ).start()
            return carry

        lax.fori_loop(0, _EMBED_DIM * _NCHUNK, fire, 0)
        pltpu.make_async_copy(
            out_hbm.at[pl.ds(wid * _NCHUNK, _NCHUNK)], slab_v, sem).wait()
        pltpu.sync_copy(slab_v, out_hbm.at[pl.ds(wid * _NCHUNK, _NCHUNK)])

    return gather_kernel(tableT, idx3)


_BC = 4096
_TPB = _BC // 128


def _mlp1_body(featT_ref, w1T_ref, b1c_ref, xf_ref):
    w1fT = w1T_ref[:, _EMBED_DIM:]
    xf = jnp.dot(w1fT, featT_ref[...],
                 preferred_element_type=jnp.float32) + b1c_ref[...]
    xf_ref[...] = xf.astype(jnp.bfloat16)


def _tc_mlp1(featT, w1T, b1c):
    grid = (_BATCH // _BC,)
    return pl.pallas_call(
        _mlp1_body,
        grid=grid,
        in_specs=[
            pl.BlockSpec((_INPUT_DIM, _BC), lambda i: (0, i)),
            pl.BlockSpec((_INPUT_DIM, _EMBED_DIM + _INPUT_DIM),
                         lambda i: (0, 0)),
            pl.BlockSpec((_HIDDEN, 1), lambda i: (0, 0)),
        ],
        out_specs=pl.BlockSpec((_HIDDEN, _BC), lambda i: (0, i)),
        out_shape=jax.ShapeDtypeStruct((_HIDDEN, _BATCH), jnp.bfloat16),
    )(featT, w1T, b1c)


def _mlp2_body(xf_ref, e_ref, w1T_ref, w2c_ref, b2_ref, out_ref):
    w1eT = w1T_ref[:, 0:_EMBED_DIM]
    e = e_ref[...]
    xe = jnp.concatenate(
        [jnp.dot(w1eT, e[t], preferred_element_type=jnp.float32)
         for t in range(_TPB)], axis=1)
    hT = jnp.maximum(xf_ref[...].astype(jnp.float32) + xe, 0.0)
    o = jnp.sum(hT * w2c_ref[...], axis=0) + b2_ref[0, 0]
    out_ref[...] = jnp.concatenate(
        [o.reshape(1, _BC), jnp.zeros((7, _BC), jnp.float32)], axis=0)


def _tc_mlp2(xfT, E, w1T, w2c, b2r):
    grid = (_BATCH // _BC,)
    return pl.pallas_call(
        _mlp2_body,
        grid=grid,
        in_specs=[
            pl.BlockSpec((_HIDDEN, _BC), lambda i: (0, i)),
            pl.BlockSpec((_TPB, _EMBED_DIM, _CHUNK), lambda i: (i, 0, 0)),
            pl.BlockSpec((_INPUT_DIM, _EMBED_DIM + _INPUT_DIM),
                         lambda i: (0, 0)),
            pl.BlockSpec((_HIDDEN, 1), lambda i: (0, 0)),
            pl.BlockSpec((1, 1), lambda i: (0, 0)),
        ],
        out_specs=pl.BlockSpec((8, _BC), lambda i: (0, i)),
        out_shape=jax.ShapeDtypeStruct((8, _BATCH), jnp.float32),
    )(xfT, E, w1T, w2c, b2r)


def kernel(pitcher_id, features, table, W1, b1, W2, b2):
    pid = pitcher_id.astype(jnp.int32)
    idx3 = pid.reshape(_NW, _NCHUNK, _CHUNK)
    w1T = W1.T
    E = _sc_gather(table.T, idx3)
    xfT = _tc_mlp1(features.T, w1T, b1.reshape(_HIDDEN, 1))
    out8 = _tc_mlp2(xfT, E, w1T, W2, b2.reshape(1, 1))
    return out8[:1, :].reshape(_BATCH, 1)


# speedup vs baseline: 1.0167x; 1.0148x over previous
import functools

import jax
import jax.numpy as jnp
from jax import lax
from jax.experimental import pallas as pl
from jax.experimental.pallas import tpu as pltpu
from jax.experimental.pallas import tpu_sc as plsc

_EMBED_DIM = 16
_INPUT_DIM = 64
_HIDDEN = 64
_BATCH = 16384
_NT = _BATCH // 128

_NC = 2
_NS = 16
_NW = _NC * _NS
_BPW = _BATCH // _NW
_CHUNK = 128
_NCHUNK = _BPW // _CHUNK


def _sc_gather(tableT, idx3):
    mesh = plsc.VectorSubcoreMesh(core_axis_name="c", subcore_axis_name="s")

    @functools.partial(
        pl.kernel,
        mesh=mesh,
        compiler_params=pltpu.CompilerParams(use_tc_tiling_on_sc=False,
                                             needs_layout_passes=False),
        out_type=jax.ShapeDtypeStruct((_NT, _EMBED_DIM, _CHUNK), jnp.float32),
        scratch_types=[
            pltpu.VMEM((_NCHUNK, _CHUNK), jnp.int32),
            pltpu.VMEM((_NCHUNK, _EMBED_DIM, _CHUNK), jnp.float32),
            pltpu.SemaphoreType.DMA,
        ],
    )
    def gather_kernel(table_hbm, idx_hbm, out_hbm, idx_v, slab_v, sem):
        wid = lax.axis_index("s") * _NC + lax.axis_index("c")
        pltpu.sync_copy(idx_hbm.at[wid], idx_v)

        def fire(i, carry):
            j = lax.shift_right_logical(i, 2)
            c = i & 3
            pltpu.make_async_copy(
                table_hbm.at[j].at[idx_v.at[c]],
                slab_v.at[c, j],
                sem,
            ).start()
            return carry

        lax.fori_loop(0, _EMBED_DIM * _NCHUNK, fire, 0)
        pltpu.make_async_copy(
            out_hbm.at[pl.ds(wid * _NCHUNK, _NCHUNK)], slab_v, sem).wait()
        pltpu.sync_copy(slab_v, out_hbm.at[pl.ds(wid * _NCHUNK, _NCHUNK)])

    return gather_kernel(tableT, idx3)


_BC = 8192
_TPB = _BC // 128


def _mlp1_body(featT_ref, w1T_ref, b1c_ref, xf_ref):
    w1fT = w1T_ref[:, _EMBED_DIM:]
    xf = jnp.dot(w1fT, featT_ref[...],
                 preferred_element_type=jnp.float32) + b1c_ref[...]
    xf_ref[...] = xf.astype(jnp.bfloat16)


def _tc_mlp1(featT, w1T, b1c):
    grid = (_BATCH // _BC,)
    return pl.pallas_call(
        _mlp1_body,
        grid=grid,
        in_specs=[
            pl.BlockSpec((_INPUT_DIM, _BC), lambda i: (0, i)),
            pl.BlockSpec((_INPUT_DIM, _EMBED_DIM + _INPUT_DIM),
                         lambda i: (0, 0)),
            pl.BlockSpec((_HIDDEN, 1), lambda i: (0, 0)),
        ],
        out_specs=pl.BlockSpec((_HIDDEN, _BC), lambda i: (0, i)),
        out_shape=jax.ShapeDtypeStruct((_HIDDEN, _BATCH), jnp.bfloat16),
    )(featT, w1T, b1c)


def _mlp2_body(xf_ref, e_ref, w1T_ref, w2c_ref, b2_ref, out_ref):
    w1eT = w1T_ref[:, 0:_EMBED_DIM]
    e = e_ref[...]
    xe = jnp.concatenate(
        [jnp.dot(w1eT, e[t], preferred_element_type=jnp.float32)
         for t in range(_TPB)], axis=1)
    hT = jnp.maximum(xf_ref[...].astype(jnp.float32) + xe, 0.0)
    o = jnp.sum(hT * w2c_ref[...], axis=0) + b2_ref[0, 0]
    out_ref[...] = jnp.concatenate(
        [o.reshape(1, _BC), jnp.zeros((7, _BC), jnp.float32)], axis=0)


def _tc_mlp2(xfT, E, w1T, w2c, b2r):
    grid = (_BATCH // _BC,)
    return pl.pallas_call(
        _mlp2_body,
        grid=grid,
        in_specs=[
            pl.BlockSpec((_HIDDEN, _BC), lambda i: (0, i)),
            pl.BlockSpec((_TPB, _EMBED_DIM, _CHUNK), lambda i: (i, 0, 0)),
            pl.BlockSpec((_INPUT_DIM, _EMBED_DIM + _INPUT_DIM),
                         lambda i: (0, 0)),
            pl.BlockSpec((_HIDDEN, 1), lambda i: (0, 0)),
            pl.BlockSpec((1, 1), lambda i: (0, 0)),
        ],
        out_specs=pl.BlockSpec((8, _BC), lambda i: (0, i)),
        out_shape=jax.ShapeDtypeStruct((8, _BATCH), jnp.float32),
    )(xfT, E, w1T, w2c, b2r)


def kernel(pitcher_id, features, table, W1, b1, W2, b2):
    pid = pitcher_id.astype(jnp.int32)
    idx3 = pid.reshape(_NW, _NCHUNK, _CHUNK)
    w1T = W1.T
    E = _sc_gather(table.T, idx3)
    xfT = _tc_mlp1(features.T, w1T, b1.reshape(_HIDDEN, 1))
    out8 = _tc_mlp2(xfT, E, w1T, W2, b2.reshape(1, 1))
    return out8[:1, :].reshape(_BATCH, 1)
